# baseline (speedup 1.0000x reference)
import functools

import jax
import jax.numpy as jnp
from jax.experimental import pallas as pl
from jax.experimental.pallas import tpu as pltpu

NE = 64
KTOP = 2
H = 2048
F = 768
NEP = 128
RB = 64
PMAX = 8192
TOK_BLK = 256


def _router_body(x_ref, g_ref, eid_ref, w_ref):
    x = x_ref[...]
    logits = jnp.dot(x, g_ref[...], preferred_element_type=jnp.float32)
    col = jax.lax.broadcasted_iota(jnp.int32, logits.shape, 1)
    neg = jnp.float32(-1e30)
    logits = jnp.where(col < NE, logits, neg)
    m = jnp.max(logits, axis=1, keepdims=True)
    e = jnp.exp(logits - m)
    p = e / jnp.sum(e, axis=1, keepdims=True)
    m1 = jnp.max(p, axis=1, keepdims=True)
    a1 = jnp.min(jnp.where(p == m1, col, NEP), axis=1)
    p2 = jnp.where(col == a1[:, None], neg, p)
    m2 = jnp.max(p2, axis=1, keepdims=True)
    a2 = jnp.min(jnp.where(p2 == m2, col, NEP), axis=1)
    s12 = m1[:, 0] + m2[:, 0]
    eid_ref[0, :] = a1
    eid_ref[1, :] = a2
    w_ref[0, :] = m1[:, 0] / s12
    w_ref[1, :] = m2[:, 0] / s12


def _run_router(hs2d, gate_kernel, tokens):
    gate_pad = jnp.zeros((H, NEP), jnp.float32).at[:, :NE].set(gate_kernel)
    grid = (tokens // TOK_BLK,)
    eid, w = pl.pallas_call(
        _router_body,
        grid=grid,
        in_specs=[
            pl.BlockSpec((TOK_BLK, H), lambda i: (i, 0)),
            pl.BlockSpec((H, NEP), lambda i: (0, 0)),
        ],
        out_specs=[
            pl.BlockSpec((8, TOK_BLK), lambda i: (0, i)),
            pl.BlockSpec((8, TOK_BLK), lambda i: (0, i)),
        ],
        out_shape=[
            jax.ShapeDtypeStruct((8, tokens), jnp.int32),
            jax.ShapeDtypeStruct((8, tokens), jnp.float32),
        ],
    )(hs2d, gate_pad)
    return eid[:KTOP], w[:KTOP]


def _ffn_body(poff_ref, wg_ref, wu_ref, wd_ref, xs_hbm, out_hbm,
              xv, yv, sin, sout):
    e = pl.program_id(0)
    start = pl.multiple_of(poff_ref[e], RB)
    n = (poff_ref[e + 1] - poff_ref[e]) // RB

    def in_copy(j, slot):
        return pltpu.make_async_copy(
            xs_hbm.at[pl.ds(start + j * RB, RB)], xv.at[slot], sin.at[slot])

    def out_copy(j, slot):
        return pltpu.make_async_copy(
            yv.at[slot], out_hbm.at[pl.ds(start + j * RB, RB)], sout.at[slot])

    @pl.when(n > 0)
    def _():
        in_copy(0, 0).start()

    def body(j, _):
        slot = jax.lax.rem(j, 2)

        @pl.when(j + 1 < n)
        def _():
            in_copy(j + 1, 1 - slot).start()

        in_copy(j, slot).wait()
        x = xv[slot]
        g = jnp.dot(x, wg_ref[0], preferred_element_type=jnp.float32)
        u = jnp.dot(x, wu_ref[0], preferred_element_type=jnp.float32)
        a = g * jax.nn.sigmoid(g) * u
        y = jnp.dot(a, wd_ref[0], preferred_element_type=jnp.float32)

        @pl.when(j >= 2)
        def _():
            out_copy(j - 2, slot).wait()

        yv[slot] = y
        out_copy(j, slot).start()
        return 0

    jax.lax.fori_loop(0, n, body, 0)

    @pl.when(n >= 2)
    def _():
        out_copy(n - 2, jax.lax.rem(n - 2, 2)).wait()

    @pl.when(n >= 1)
    def _():
        out_copy(n - 1, jax.lax.rem(n - 1, 2)).wait()


def _run_ffn(xs, poff, gate_proj, up_proj, down_proj):
    grid_spec = pltpu.PrefetchScalarGridSpec(
        num_scalar_prefetch=1,
        grid=(NE,),
        in_specs=[
            pl.BlockSpec((1, H, F), lambda e, poff: (e, 0, 0)),
            pl.BlockSpec((1, H, F), lambda e, poff: (e, 0, 0)),
            pl.BlockSpec((1, F, H), lambda e, poff: (e, 0, 0)),
            pl.BlockSpec(memory_space=pl.ANY),
        ],
        out_specs=pl.BlockSpec(memory_space=pl.ANY),
        scratch_shapes=[
            pltpu.VMEM((2, RB, H), jnp.float32),
            pltpu.VMEM((2, RB, H), jnp.float32),
            pltpu.SemaphoreType.DMA((2,)),
            pltpu.SemaphoreType.DMA((2,)),
        ],
    )
    return pl.pallas_call(
        _ffn_body,
        grid_spec=grid_spec,
        out_shape=jax.ShapeDtypeStruct((PMAX, H), jnp.float32),
    )(poff, gate_proj, up_proj, down_proj, xs)


def kernel(hidden_states, gate_kernel, gate_proj, up_proj, down_proj):
    b, s, _ = hidden_states.shape
    tokens = b * s
    hs2d = hidden_states.reshape(tokens, H)

    eid, w = _run_router(hs2d, gate_kernel, tokens)
    flat_sel = jnp.stack([eid[0], eid[1]], axis=1).reshape(-1)
    nc = tokens * KTOP

    sizes = jnp.zeros((NE,), jnp.int32).at[flat_sel].add(1)
    cap = (sizes + (RB - 1)) // RB * RB
    poff = jnp.concatenate([jnp.zeros((1,), jnp.int32),
                            jnp.cumsum(cap).astype(jnp.int32)])
    uoff = jnp.concatenate([jnp.zeros((1,), jnp.int32),
                            jnp.cumsum(sizes).astype(jnp.int32)])
    order = jnp.argsort(flat_sel, stable=True).astype(jnp.int32)
    ej = flat_sel[order]
    pslot = poff[ej] + (jnp.arange(nc, dtype=jnp.int32) - uoff[ej])
    pos = jnp.zeros((nc,), jnp.int32).at[order].set(pslot)
    perm = jnp.zeros((PMAX,), jnp.int32).at[pslot].set(order // KTOP)

    xs = jnp.take(hs2d, perm, axis=0)
    ys = _run_ffn(xs, poff, gate_proj, up_proj, down_proj)

    y0 = jnp.take(ys, pos[0::2], axis=0)
    y1 = jnp.take(ys, pos[1::2], axis=0)
    out = w[0][:, None] * y0 + w[1][:, None] * y1
    return out.reshape(b, s, H)


# speedup vs baseline: 1.0410x; 1.0410x over previous
import functools

import jax
import jax.numpy as jnp
from jax.experimental import pallas as pl
from jax.experimental.pallas import tpu as pltpu

NE = 64
KTOP = 2
H = 2048
F = 768
NEP = 128
RB = 64
PMAX = 8192
TOK_BLK = 256


def _router_body(x_ref, g_ref, eid_ref, w_ref):
    x = x_ref[...]
    logits = jnp.dot(x, g_ref[...], preferred_element_type=jnp.float32)
    col = jax.lax.broadcasted_iota(jnp.int32, logits.shape, 1)
    neg = jnp.float32(-1e30)
    logits = jnp.where(col < NE, logits, neg)
    m = jnp.max(logits, axis=1, keepdims=True)
    e = jnp.exp(logits - m)
    p = e / jnp.sum(e, axis=1, keepdims=True)
    m1 = jnp.max(p, axis=1, keepdims=True)
    a1 = jnp.min(jnp.where(p == m1, col, NEP), axis=1)
    p2 = jnp.where(col == a1[:, None], neg, p)
    m2 = jnp.max(p2, axis=1, keepdims=True)
    a2 = jnp.min(jnp.where(p2 == m2, col, NEP), axis=1)
    s12 = m1[:, 0] + m2[:, 0]
    eid_ref[0, :] = a1
    eid_ref[1, :] = a2
    w_ref[0, :] = m1[:, 0] / s12
    w_ref[1, :] = m2[:, 0] / s12


def _run_router(hs2d, gate_kernel, tokens):
    gate_pad = jnp.zeros((H, NEP), jnp.float32).at[:, :NE].set(gate_kernel)
    grid = (tokens // TOK_BLK,)
    eid, w = pl.pallas_call(
        _router_body,
        grid=grid,
        in_specs=[
            pl.BlockSpec((TOK_BLK, H), lambda i: (i, 0)),
            pl.BlockSpec((H, NEP), lambda i: (0, 0)),
        ],
        out_specs=[
            pl.BlockSpec((8, TOK_BLK), lambda i: (0, i)),
            pl.BlockSpec((8, TOK_BLK), lambda i: (0, i)),
        ],
        out_shape=[
            jax.ShapeDtypeStruct((8, tokens), jnp.int32),
            jax.ShapeDtypeStruct((8, tokens), jnp.float32),
        ],
    )(hs2d, gate_pad)
    return eid[:KTOP], w[:KTOP]


def _ffn_body(poff_ref, wg_ref, wu_ref, wd_ref, xs_hbm, out_hbm,
              xv, yv, sin, sout):
    e = pl.program_id(0)
    start = pl.multiple_of(poff_ref[e], RB)
    n = (poff_ref[e + 1] - poff_ref[e]) // RB

    def in_copy(j, slot):
        return pltpu.make_async_copy(
            xs_hbm.at[pl.ds(start + j * RB, RB)], xv.at[slot], sin.at[slot])

    def out_copy(j, slot):
        return pltpu.make_async_copy(
            yv.at[slot], out_hbm.at[pl.ds(start + j * RB, RB)], sout.at[slot])

    @pl.when(n > 0)
    def _():
        in_copy(0, 0).start()

    def body(j, _):
        slot = jax.lax.rem(j, 2)

        @pl.when(j + 1 < n)
        def _():
            in_copy(j + 1, 1 - slot).start()

        in_copy(j, slot).wait()
        x = xv[slot]
        g = jnp.dot(x, wg_ref[0], preferred_element_type=jnp.float32)
        u = jnp.dot(x, wu_ref[0], preferred_element_type=jnp.float32)
        a = g * jax.nn.sigmoid(g) * u
        y = jnp.dot(a, wd_ref[0], preferred_element_type=jnp.float32)

        @pl.when(j >= 2)
        def _():
            out_copy(j - 2, slot).wait()

        yv[slot] = y
        out_copy(j, slot).start()
        return 0

    jax.lax.fori_loop(0, n, body, 0)

    @pl.when(n >= 2)
    def _():
        out_copy(n - 2, jax.lax.rem(n - 2, 2)).wait()

    @pl.when(n >= 1)
    def _():
        out_copy(n - 1, jax.lax.rem(n - 1, 2)).wait()


def _run_ffn(xs, poff, gate_proj, up_proj, down_proj):
    grid_spec = pltpu.PrefetchScalarGridSpec(
        num_scalar_prefetch=1,
        grid=(NE,),
        in_specs=[
            pl.BlockSpec((1, H, F), lambda e, poff: (e, 0, 0)),
            pl.BlockSpec((1, H, F), lambda e, poff: (e, 0, 0)),
            pl.BlockSpec((1, F, H), lambda e, poff: (e, 0, 0)),
            pl.BlockSpec(memory_space=pl.ANY),
        ],
        out_specs=pl.BlockSpec(memory_space=pl.ANY),
        scratch_shapes=[
            pltpu.VMEM((2, RB, H), jnp.float32),
            pltpu.VMEM((2, RB, H), jnp.float32),
            pltpu.SemaphoreType.DMA((2,)),
            pltpu.SemaphoreType.DMA((2,)),
        ],
    )
    return pl.pallas_call(
        _ffn_body,
        grid_spec=grid_spec,
        out_shape=jax.ShapeDtypeStruct((PMAX, H), jnp.float32),
    )(poff, gate_proj, up_proj, down_proj, xs)


def kernel(hidden_states, gate_kernel, gate_proj, up_proj, down_proj):
    b, s, _ = hidden_states.shape
    tokens = b * s
    hs2d = hidden_states.reshape(tokens, H)

    eid, w = _run_router(hs2d, gate_kernel, tokens)
    flat_sel = jnp.stack([eid[0], eid[1]], axis=1).reshape(-1)
    nc = tokens * KTOP

    onehot = (flat_sel[:, None] == jnp.arange(NE, dtype=jnp.int32)[None, :])
    onehot = onehot.astype(jnp.int32)
    csum = jnp.cumsum(onehot, axis=0)
    sizes = csum[-1]
    cap = (sizes + (RB - 1)) // RB * RB
    poff = jnp.concatenate([jnp.zeros((1,), jnp.int32),
                            jnp.cumsum(cap).astype(jnp.int32)])
    rank = jnp.take_along_axis(csum, flat_sel[:, None], axis=1)[:, 0] - 1
    pos = poff[flat_sel] + rank
    perm = jnp.zeros((PMAX,), jnp.int32).at[pos].set(
        jnp.arange(nc, dtype=jnp.int32) // KTOP)

    xs = jnp.take(hs2d, perm, axis=0)
    ys = _run_ffn(xs, poff, gate_proj, up_proj, down_proj)

    y0 = jnp.take(ys, pos[0::2], axis=0)
    y1 = jnp.take(ys, pos[1::2], axis=0)
    out = w[0][:, None] * y0 + w[1][:, None] * y1
    return out.reshape(b, s, H)


# speedup vs baseline: 1.2587x; 1.2092x over previous
import functools

import jax
import jax.numpy as jnp
from jax.experimental import pallas as pl
from jax.experimental.pallas import tpu as pltpu

NE = 64
KTOP = 2
H = 2048
F = 768
NEP = 128
RB = 64
PMAX = 8192
TOK_BLK = 256


def _router_body(x_ref, g_ref, eid_ref, w_ref):
    x = x_ref[...]
    logits = jnp.dot(x, g_ref[...], preferred_element_type=jnp.float32)
    col = jax.lax.broadcasted_iota(jnp.int32, logits.shape, 1)
    neg = jnp.float32(-1e30)
    logits = jnp.where(col < NE, logits, neg)
    m = jnp.max(logits, axis=1, keepdims=True)
    e = jnp.exp(logits - m)
    p = e / jnp.sum(e, axis=1, keepdims=True)
    m1 = jnp.max(p, axis=1, keepdims=True)
    a1 = jnp.min(jnp.where(p == m1, col, NEP), axis=1)
    p2 = jnp.where(col == a1[:, None], neg, p)
    m2 = jnp.max(p2, axis=1, keepdims=True)
    a2 = jnp.min(jnp.where(p2 == m2, col, NEP), axis=1)
    s12 = m1[:, 0] + m2[:, 0]
    eid_ref[0, :] = a1
    eid_ref[1, :] = a2
    w_ref[0, :] = m1[:, 0] / s12
    w_ref[1, :] = m2[:, 0] / s12


def _run_router(hs2d, gate_kernel, tokens):
    gate_pad = jnp.zeros((H, NEP), jnp.float32).at[:, :NE].set(gate_kernel)
    grid = (tokens // TOK_BLK,)
    eid, w = pl.pallas_call(
        _router_body,
        grid=grid,
        in_specs=[
            pl.BlockSpec((TOK_BLK, H), lambda i: (i, 0)),
            pl.BlockSpec((H, NEP), lambda i: (0, 0)),
        ],
        out_specs=[
            pl.BlockSpec((8, TOK_BLK), lambda i: (0, i)),
            pl.BlockSpec((8, TOK_BLK), lambda i: (0, i)),
        ],
        out_shape=[
            jax.ShapeDtypeStruct((8, tokens), jnp.int32),
            jax.ShapeDtypeStruct((8, tokens), jnp.float32),
        ],
    )(hs2d, gate_pad)
    return eid[:KTOP], w[:KTOP]


def _ffn_body(poff_ref, wg_ref, wu_ref, wd_ref, xs_hbm, out_hbm,
              xv, yv, sin, sout):
    e = pl.program_id(0)
    start = pl.multiple_of(poff_ref[e], RB)
    n = (poff_ref[e + 1] - poff_ref[e]) // RB
    base = start // RB

    def slot_of(j):
        return jax.lax.rem(base + j, 2)

    def in_copy(j, slot):
        return pltpu.make_async_copy(
            xs_hbm.at[pl.ds(start + j * RB, RB)], xv.at[slot], sin.at[slot])

    def out_copy(j, slot):
        return pltpu.make_async_copy(
            yv.at[slot], out_hbm.at[pl.ds(start + j * RB, RB)], sout.at[slot])

    @pl.when((e == 0) & (n > 0))
    def _():
        in_copy(0, slot_of(0)).start()

    def body(j, _):
        slot = slot_of(j)

        @pl.when(j + 1 < n)
        def _():
            in_copy(j + 1, 1 - slot).start()

        in_copy(j, slot).wait()
        x = xv[slot]
        g = jnp.dot(x, wg_ref[0], preferred_element_type=jnp.float32)
        u = jnp.dot(x, wu_ref[0], preferred_element_type=jnp.float32)
        a = g * jax.nn.sigmoid(g) * u
        y = jnp.dot(a, wd_ref[0], preferred_element_type=jnp.float32)

        @pl.when(j >= 2)
        def _():
            out_copy(j - 2, slot).wait()

        yv[slot] = y
        out_copy(j, slot).start()
        return 0

    jax.lax.fori_loop(0, n, body, 0)

    @pl.when(e + 1 < NE)
    def _():
        nn = (poff_ref[e + 2] - poff_ref[e + 1]) // RB

        @pl.when(nn > 0)
        def _():
            nstart = pl.multiple_of(poff_ref[e + 1], RB)
            pltpu.make_async_copy(
                xs_hbm.at[pl.ds(nstart, RB)],
                xv.at[slot_of(n)], sin.at[slot_of(n)]).start()

    @pl.when(n >= 2)
    def _():
        out_copy(n - 2, slot_of(n - 2)).wait()

    @pl.when(n >= 1)
    def _():
        out_copy(n - 1, slot_of(n - 1)).wait()


def _run_ffn(xs, poff, gate_proj, up_proj, down_proj):
    grid_spec = pltpu.PrefetchScalarGridSpec(
        num_scalar_prefetch=1,
        grid=(NE,),
        in_specs=[
            pl.BlockSpec((1, H, F), lambda e, poff: (e, 0, 0)),
            pl.BlockSpec((1, H, F), lambda e, poff: (e, 0, 0)),
            pl.BlockSpec((1, F, H), lambda e, poff: (e, 0, 0)),
            pl.BlockSpec(memory_space=pl.ANY),
        ],
        out_specs=pl.BlockSpec(memory_space=pl.ANY),
        scratch_shapes=[
            pltpu.VMEM((2, RB, H), jnp.float32),
            pltpu.VMEM((2, RB, H), jnp.float32),
            pltpu.SemaphoreType.DMA((2,)),
            pltpu.SemaphoreType.DMA((2,)),
        ],
    )
    return pl.pallas_call(
        _ffn_body,
        grid_spec=grid_spec,
        out_shape=jax.ShapeDtypeStruct((PMAX, H), jnp.float32),
    )(poff, gate_proj, up_proj, down_proj, xs)


def kernel(hidden_states, gate_kernel, gate_proj, up_proj, down_proj):
    b, s, _ = hidden_states.shape
    tokens = b * s
    hs2d = hidden_states.reshape(tokens, H)

    eid, w = _run_router(hs2d, gate_kernel, tokens)
    flat_sel = jnp.stack([eid[0], eid[1]], axis=1).reshape(-1)
    nc = tokens * KTOP

    onehot = (flat_sel[:, None] == jnp.arange(NE, dtype=jnp.int32)[None, :])
    onehot = onehot.astype(jnp.int32)
    csum = jnp.cumsum(onehot, axis=0)
    sizes = csum[-1]
    cap = (sizes + (RB - 1)) // RB * RB
    poff = jnp.concatenate([jnp.zeros((1,), jnp.int32),
                            jnp.cumsum(cap).astype(jnp.int32)])
    rank = jnp.take_along_axis(csum, flat_sel[:, None], axis=1)[:, 0] - 1
    pos = poff[flat_sel] + rank
    perm = jnp.zeros((PMAX,), jnp.int32).at[pos].set(
        jnp.arange(nc, dtype=jnp.int32) // KTOP)

    xs = jnp.take(hs2d, perm, axis=0)
    ys = _run_ffn(xs, poff, gate_proj, up_proj, down_proj)

    y0 = jnp.take(ys, pos[0::2], axis=0)
    y1 = jnp.take(ys, pos[1::2], axis=0)
    out = w[0][:, None] * y0 + w[1][:, None] * y1
    return out.reshape(b, s, H)


# speedup vs baseline: 1.2852x; 1.0211x over previous
import functools

import jax
import jax.numpy as jnp
from jax.experimental import pallas as pl
from jax.experimental.pallas import tpu as pltpu

NE = 64
KTOP = 2
H = 2048
F = 768
NEP = 128
RB = 64
PMAX = 8192
TOK_BLK = 256


def _router_body(x_ref, g_ref, eid_ref, w_ref):
    x = x_ref[...]
    logits = jnp.dot(x, g_ref[...], preferred_element_type=jnp.float32)
    col = jax.lax.broadcasted_iota(jnp.int32, logits.shape, 1)
    neg = jnp.float32(-1e30)
    logits = jnp.where(col < NE, logits, neg)
    m = jnp.max(logits, axis=1, keepdims=True)
    e = jnp.exp(logits - m)
    p = e / jnp.sum(e, axis=1, keepdims=True)
    m1 = jnp.max(p, axis=1, keepdims=True)
    a1 = jnp.min(jnp.where(p == m1, col, NEP), axis=1)
    p2 = jnp.where(col == a1[:, None], neg, p)
    m2 = jnp.max(p2, axis=1, keepdims=True)
    a2 = jnp.min(jnp.where(p2 == m2, col, NEP), axis=1)
    s12 = m1[:, 0] + m2[:, 0]
    eid_ref[0, :] = a1
    eid_ref[1, :] = a2
    w_ref[0, :] = m1[:, 0] / s12
    w_ref[1, :] = m2[:, 0] / s12


def _run_router(hs2d, gate_kernel, tokens):
    gate_pad = jnp.zeros((H, NEP), jnp.float32).at[:, :NE].set(gate_kernel)
    grid = (tokens // TOK_BLK,)
    eid, w = pl.pallas_call(
        _router_body,
        grid=grid,
        in_specs=[
            pl.BlockSpec((TOK_BLK, H), lambda i: (i, 0)),
            pl.BlockSpec((H, NEP), lambda i: (0, 0)),
        ],
        out_specs=[
            pl.BlockSpec((8, TOK_BLK), lambda i: (0, i)),
            pl.BlockSpec((8, TOK_BLK), lambda i: (0, i)),
        ],
        out_shape=[
            jax.ShapeDtypeStruct((8, tokens), jnp.int32),
            jax.ShapeDtypeStruct((8, tokens), jnp.float32),
        ],
    )(hs2d, gate_pad)
    return eid[:KTOP], w[:KTOP]


def _ffn_body(poff_ref, wg_ref, wu_ref, wd_ref, xs_hbm, out_hbm,
              xv, yv, sin, sout):
    e = pl.program_id(0)
    start = pl.multiple_of(poff_ref[e], RB)
    n = (poff_ref[e + 1] - poff_ref[e]) // RB
    base = start // RB

    def slot_of(j):
        return jax.lax.rem(base + j, 2)

    def in_copy(j, slot):
        return pltpu.make_async_copy(
            xs_hbm.at[pl.ds(start + j * RB, RB)], xv.at[slot], sin.at[slot])

    def out_copy(g):
        slot = jax.lax.rem(g, 2)
        return pltpu.make_async_copy(
            yv.at[slot], out_hbm.at[pl.ds(g * RB, RB)], sout.at[slot])

    @pl.when((e == 0) & (n > 0))
    def _():
        in_copy(0, slot_of(0)).start()

    def body(j, _):
        slot = slot_of(j)

        @pl.when(j + 1 < n)
        def _():
            in_copy(j + 1, 1 - slot).start()

        in_copy(j, slot).wait()
        x = xv[slot]
        g = jnp.dot(x, wg_ref[0], preferred_element_type=jnp.float32)
        u = jnp.dot(x, wu_ref[0], preferred_element_type=jnp.float32)
        a = g * jax.nn.sigmoid(g) * u
        y = jnp.dot(a, wd_ref[0], preferred_element_type=jnp.float32)

        gj = base + j

        @pl.when(gj >= 2)
        def _():
            out_copy(gj - 2).wait()

        yv[slot] = y
        out_copy(gj).start()
        return 0

    jax.lax.fori_loop(0, n, body, 0)

    @pl.when(e + 1 < NE)
    def _():
        nn = (poff_ref[e + 2] - poff_ref[e + 1]) // RB

        @pl.when(nn > 0)
        def _():
            nstart = pl.multiple_of(poff_ref[e + 1], RB)
            pltpu.make_async_copy(
                xs_hbm.at[pl.ds(nstart, RB)],
                xv.at[slot_of(n)], sin.at[slot_of(n)]).start()

    @pl.when(e == NE - 1)
    def _():
        gtot = poff_ref[NE] // RB

        @pl.when(gtot >= 2)
        def _():
            out_copy(gtot - 2).wait()

        @pl.when(gtot >= 1)
        def _():
            out_copy(gtot - 1).wait()


def _run_ffn(xs, poff, gate_proj, up_proj, down_proj):
    grid_spec = pltpu.PrefetchScalarGridSpec(
        num_scalar_prefetch=1,
        grid=(NE,),
        in_specs=[
            pl.BlockSpec((1, H, F), lambda e, poff: (e, 0, 0)),
            pl.BlockSpec((1, H, F), lambda e, poff: (e, 0, 0)),
            pl.BlockSpec((1, F, H), lambda e, poff: (e, 0, 0)),
            pl.BlockSpec(memory_space=pl.ANY),
        ],
        out_specs=pl.BlockSpec(memory_space=pl.ANY),
        scratch_shapes=[
            pltpu.VMEM((2, RB, H), jnp.float32),
            pltpu.VMEM((2, RB, H), jnp.float32),
            pltpu.SemaphoreType.DMA((2,)),
            pltpu.SemaphoreType.DMA((2,)),
        ],
    )
    return pl.pallas_call(
        _ffn_body,
        grid_spec=grid_spec,
        out_shape=jax.ShapeDtypeStruct((PMAX, H), jnp.float32),
    )(poff, gate_proj, up_proj, down_proj, xs)


def kernel(hidden_states, gate_kernel, gate_proj, up_proj, down_proj):
    b, s, _ = hidden_states.shape
    tokens = b * s
    hs2d = hidden_states.reshape(tokens, H)

    eid, w = _run_router(hs2d, gate_kernel, tokens)
    flat_sel = jnp.stack([eid[0], eid[1]], axis=1).reshape(-1)
    nc = tokens * KTOP

    onehot = (flat_sel[:, None] == jnp.arange(NE, dtype=jnp.int32)[None, :])
    onehot = onehot.astype(jnp.int32)
    csum = jnp.cumsum(onehot, axis=0)
    sizes = csum[-1]
    cap = (sizes + (RB - 1)) // RB * RB
    poff = jnp.concatenate([jnp.zeros((1,), jnp.int32),
                            jnp.cumsum(cap).astype(jnp.int32)])
    rank = jnp.take_along_axis(csum, flat_sel[:, None], axis=1)[:, 0] - 1
    pos = poff[flat_sel] + rank
    perm = jnp.zeros((PMAX,), jnp.int32).at[pos].set(
        jnp.arange(nc, dtype=jnp.int32) // KTOP)

    xs = jnp.take(hs2d, perm, axis=0)
    ys = _run_ffn(xs, poff, gate_proj, up_proj, down_proj)

    y0 = jnp.take(ys, pos[0::2], axis=0)
    y1 = jnp.take(ys, pos[1::2], axis=0)
    out = w[0][:, None] * y0 + w[1][:, None] * y1
    return out.reshape(b, s, H)
